# Initial kernel scaffold; baseline (speedup 1.0000x reference)
import jax
import jax.numpy as jnp
from jax.experimental import pallas as pl


def kernel(x, edge_index, W, b):
    raise NotImplementedError("write your pallas kernel here")


# speedup vs baseline: 18.2941x; 18.2941x over previous
import functools

import jax
import jax.numpy as jnp
from jax import lax
from jax.experimental import pallas as pl
from jax.experimental.pallas import tpu as pltpu
from jax.experimental.pallas import tpu_sc as plsc

N_NODES = 10000
N_EDGES = 320000
D = 128

NC = 2
NS = 16
NW = NC * NS

EPT = N_EDGES // NW
K = 80
NCHUNK = EPT // K

H_BINS = 10240
H_R = 80
A_ROWS_T = 632
A_ROWS = A_ROWS_T * NS

_mesh = plsc.VectorSubcoreMesh(core_axis_name="c", subcore_axis_name="s")


def _wid():
  return lax.axis_index("s") * NC + lax.axis_index("c")


@functools.partial(
    pl.kernel,
    out_type=jax.ShapeDtypeStruct((NC, H_R, 128), jnp.float32),
    mesh=_mesh,
    scratch_types=[
        pltpu.VMEM((1, K), jnp.int32),
        pltpu.VMEM((1, H_R), jnp.int32),
        pltpu.VMEM((H_R, 128), jnp.float32),
        pltpu.VMEM_SHARED((H_R, 128), jnp.float32),
    ],
    compiler_params=pltpu.CompilerParams(needs_layout_passes=False),
)
def _deg_kernel(dst_hbm, iota_hbm, z_hbm, deg_out, idx_v, iota_v, hist_v,
                hist_sh):
  cid = lax.axis_index("c")
  sid = lax.axis_index("s")
  wid = _wid()
  pltpu.sync_copy(iota_hbm, iota_v)
  pltpu.sync_copy(z_hbm, hist_v)

  @pl.when(sid == 0)
  def _():
    pltpu.sync_copy(z_hbm, hist_sh)

  ones16 = jnp.ones((16,), jnp.float32)

  def chunk(i, carry):
    base = wid * EPT + i * K
    pltpu.sync_copy(dst_hbm.at[pl.ds(base, K)], idx_v.at[0])
    for g in range(K // 16):
      idx = idx_v[0, g * 16:(g + 1) * 16]
      row = lax.shift_right_logical(idx, 7)
      col = lax.bitwise_and(idx, 127)
      plsc.addupdate_scatter(hist_v, [row, col], ones16)
    return carry

  lax.fori_loop(0, NCHUNK, chunk, 0)
  plsc.subcore_barrier()
  pltpu.sync_copy(hist_v, hist_sh.at[iota_v.at[0]], add=True)
  plsc.subcore_barrier()

  @pl.when(sid == 0)
  def _():
    pltpu.sync_copy(hist_sh, deg_out.at[cid])


def _scale_body(degp_ref, x_ref, v_ref):
  deg = degp_ref[0, :N_NODES] + degp_ref[1, :N_NODES] + 1.0
  dinv = lax.rsqrt(deg)
  v_ref[...] = x_ref[...] * dinv[:, None]


def _scale_call(degp, x):
  return pl.pallas_call(
      _scale_body,
      out_shape=jax.ShapeDtypeStruct((N_NODES, D), jnp.float32),
  )(degp, x)


@functools.partial(
    pl.kernel,
    out_type=jax.ShapeDtypeStruct((NC, A_ROWS, D), jnp.float32),
    mesh=_mesh,
    scratch_types=[
        pltpu.VMEM((1, K), jnp.int32),
        pltpu.VMEM((1, K), jnp.int32),
        pltpu.VMEM((K, D), jnp.float32),
        pltpu.SemaphoreType.DMA,
        pltpu.VMEM_SHARED((A_ROWS, D), jnp.float32),
    ],
)
def _edge_kernel(src_hbm, dst_hbm, v_hbm, z_hbm, acc_out,
                 idxs_v, idxd_v, rows_v, sem, acc_sh):
  cid = lax.axis_index("c")
  sid = lax.axis_index("s")
  wid = _wid()
  pltpu.sync_copy(z_hbm, acc_sh.at[pl.ds(sid * A_ROWS_T, A_ROWS_T)])
  plsc.subcore_barrier()

  def chunk(i, carry):
    base = wid * EPT + i * K
    pltpu.sync_copy(src_hbm.at[pl.ds(base, K)], idxs_v.at[0])
    pltpu.sync_copy(dst_hbm.at[pl.ds(base, K)], idxd_v.at[0])
    pltpu.async_copy(v_hbm.at[idxs_v.at[0]], rows_v, sem).wait()
    pltpu.sync_copy(rows_v, acc_sh.at[idxd_v.at[0]], add=True)
    return carry

  lax.fori_loop(0, NCHUNK, chunk, 0)
  plsc.subcore_barrier()
  pltpu.sync_copy(
      acc_sh.at[pl.ds(sid * A_ROWS_T, A_ROWS_T)],
      acc_out.at[cid, pl.ds(sid * A_ROWS_T, A_ROWS_T)],
  )


def _out_body(accp_ref, degp_ref, x_ref, w_ref, b_ref, o_ref):
  deg = degp_ref[0, :N_NODES] + degp_ref[1, :N_NODES] + 1.0
  dinv = lax.rsqrt(deg)
  s = accp_ref[0, :N_NODES] + accp_ref[1, :N_NODES] + x_ref[...] * dinv[:, None]
  t = s * dinv[:, None]
  o_ref[...] = (
      jnp.dot(t, w_ref[...], preferred_element_type=jnp.float32)
      + b_ref[...][None, :]
  )


def _out_call(accp, degp, x, W, b):
  return pl.pallas_call(
      _out_body,
      out_shape=jax.ShapeDtypeStruct((N_NODES, D), jnp.float32),
  )(accp, degp, x, W, b)


def kernel(x, edge_index, W, b):
  ei = edge_index.astype(jnp.int32)
  src = ei[0]
  dst = ei[1]
  iota_r = jnp.arange(H_R, dtype=jnp.int32).reshape(1, H_R)
  z_hist = jnp.zeros((H_R, 128), jnp.float32)
  z_acc = jnp.zeros((A_ROWS_T, D), jnp.float32)

  degp = _deg_kernel(dst, iota_r, z_hist)
  degf = degp.reshape(NC, H_BINS)
  v = _scale_call(degf, x)
  accp = _edge_kernel(src, dst, v, z_acc)
  return _out_call(accp, degf, x, W, b)
